# Initial kernel scaffold; baseline (speedup 1.0000x reference)
import jax
import jax.numpy as jnp
from jax.experimental import pallas as pl


def kernel(X, C, W_node, b_node, W_edge, b_edge):
    raise NotImplementedError("write your pallas kernel here")


# speedup vs baseline: 4.8468x; 4.8468x over previous
import functools

import jax
import jax.numpy as jnp
from jax import lax
from jax.experimental import pallas as pl
from jax.experimental.pallas import tpu as pltpu
from jax.experimental.pallas import tpu_sc as plsc

B, L, A, K = 4, 1024, 4, 30
DIM_NODES, DIM_EDGES = 256, 128
RB = 256
EB = 1920
NW = 32
CHUNK = 128
EPW = (B * L * K) // NW
NCH = EPW // CHUNK
F32 = jnp.float32


def _enc_body(xt_ref, xr_ref, c_ref, wn_ref, bn_ref,
              eidx_ref, nodeh_ref, maski_ref):
    i = pl.program_id(1)
    xt = xt_ref[0]
    xr = xr_ref[0]

    cax = (((xt[0:1] + xt[3:4]) + xt[6:7]) + xt[9:10]) * 0.25
    cay = (((xt[1:2] + xt[4:5]) + xt[7:8]) + xt[10:11]) * 0.25
    caz = (((xt[2:3] + xt[5:6]) + xt[8:9]) + xt[11:12]) * 0.25
    crx = (((xr[:, 0:1] + xr[:, 3:4]) + xr[:, 6:7]) + xr[:, 9:10]) * 0.25
    cry = (((xr[:, 1:2] + xr[:, 4:5]) + xr[:, 7:8]) + xr[:, 10:11]) * 0.25
    crz = (((xr[:, 2:3] + xr[:, 5:6]) + xr[:, 8:9]) + xr[:, 11:12]) * 0.25

    dx = crx - cax
    dy = cry - cay
    dz = crz - caz
    dist = jnp.sqrt((dx * dx + dy * dy) + dz * dz + 1e-8)
    colj = lax.broadcasted_iota(jnp.int32, (RB, L), 1)
    rowg = i * RB + lax.broadcasted_iota(jnp.int32, (RB, L), 0)
    dist = dist + jnp.where(colj == rowg, F32(1e6), F32(0.0))

    lane32 = lax.broadcasted_iota(jnp.int32, (RB, 32), 1)

    def body(k, carry):
        dcur, acc = carry
        m = jnp.min(dcur, axis=1, keepdims=True)
        j = jnp.min(jnp.where(dcur == m, colj, L), axis=1, keepdims=True)
        acc = jnp.where(lane32 == k, j, acc)
        dcur = jnp.where(colj == j, F32(jnp.inf), dcur)
        return dcur, acc

    _, acc = lax.fori_loop(0, K, body, (dist, jnp.zeros((RB, 32), jnp.int32)))
    eidx_ref[0] = acc

    cols = []
    for (a, b) in ((0, 1), (0, 2), (0, 3), (1, 2), (1, 3), (2, 3)):
        pdx = xr[:, 3 * a:3 * a + 1] - xr[:, 3 * b:3 * b + 1]
        pdy = xr[:, 3 * a + 1:3 * a + 2] - xr[:, 3 * b + 1:3 * b + 2]
        pdz = xr[:, 3 * a + 2:3 * a + 3] - xr[:, 3 * b + 2:3 * b + 3]
        pd2 = (pdx * pdx + pdy * pdy) + pdz * pdz
        cols.append(jnp.log1p(jnp.sqrt(pd2 + 1e-8)))
    nf = jnp.concatenate(cols + [jnp.zeros((RB, 2), F32)], axis=1)

    mask = (c_ref[0] > 0).astype(F32)
    nh = jnp.dot(nf, wn_ref[...], preferred_element_type=F32) + bn_ref[...]
    nodeh_ref[0] = nh * mask
    maski_ref[0] = mask


def _encode_nodes(Xf, Xt, C3, Wn_pad, bn2):
    grid = (B, L // RB)
    return pl.pallas_call(
        _enc_body,
        grid=grid,
        in_specs=[
            pl.BlockSpec((1, 12, L), lambda b, i: (b, 0, 0)),
            pl.BlockSpec((1, RB, 12), lambda b, i: (b, i, 0)),
            pl.BlockSpec((1, RB, 1), lambda b, i: (b, i, 0)),
            pl.BlockSpec((8, DIM_NODES), lambda b, i: (0, 0)),
            pl.BlockSpec((1, DIM_NODES), lambda b, i: (0, 0)),
        ],
        out_specs=[
            pl.BlockSpec((1, RB, 32), lambda b, i: (b, i, 0)),
            pl.BlockSpec((1, RB, DIM_NODES), lambda b, i: (b, i, 0)),
            pl.BlockSpec((1, RB, 1), lambda b, i: (b, i, 0)),
        ],
        out_shape=[
            jax.ShapeDtypeStruct((B, L, 32), jnp.int32),
            jax.ShapeDtypeStruct((B, L, DIM_NODES), F32),
            jax.ShapeDtypeStruct((B, L, 1), F32),
        ],
    )(Xt, Xf, C3, Wn_pad, bn2)


TW = 64


def _gather_rows(table, gidx):
    mesh = plsc.VectorSubcoreMesh(core_axis_name="c", subcore_axis_name="s")

    GR = 6
    NG = NCH // GR
    GROWS = GR * CHUNK

    @functools.partial(
        pl.kernel, mesh=mesh,
        compiler_params=pltpu.CompilerParams(use_tc_tiling_on_sc=False),
        out_type=jax.ShapeDtypeStruct((NW * EPW, TW), F32),
        scratch_types=[
            pltpu.VMEM((NCH, CHUNK), jnp.int32),
            pltpu.VMEM((GROWS, TW), F32),
            pltpu.VMEM((GROWS, TW), F32),
            pltpu.SemaphoreType.DMA,
            pltpu.SemaphoreType.DMA,
            pltpu.SemaphoreType.DMA,
        ],
    )
    def k(table_hbm, idx_hbm, out_hbm, idx_v, rows0, rows1, sem_in,
          so0, so1, ):
        wid = lax.axis_index("s") * 2 + lax.axis_index("c")
        pltpu.sync_copy(idx_hbm.at[wid], idx_v)
        bufs = (rows0, rows1)
        osems = (so0, so1)
        outcp = [None, None]
        for g in range(NG):
            buf = bufs[g % 2]
            if outcp[g % 2] is not None:
                outcp[g % 2].wait()
            cps = [
                pltpu.async_copy(table_hbm.at[idx_v.at[g * GR + c]],
                                 buf.at[pl.ds(c * CHUNK, CHUNK)], sem_in)
                for c in range(GR)
            ]
            for cp in cps:
                cp.wait()
            outcp[g % 2] = pltpu.async_copy(
                buf, out_hbm.at[pl.ds(wid * EPW + g * GROWS, GROWS)],
                osems[g % 2])
        for oc in outcp:
            if oc is not None:
                oc.wait()

    return k(table, gidx)


def _edge_body(xi_ref, xj_ref, w16_ref, w17_ref, be_ref, eh_ref, mij_ref):
    xi = xi_ref[0]
    xj = xj_ref[0]

    diff = xi[:, 0:48] - xj[:, 0:48]
    dx = diff[:, 0:16]
    dy = diff[:, 16:32]
    dz = diff[:, 32:48]
    d2 = (dx * dx + dy * dy) + dz * dz
    dcr = jnp.log1p(jnp.sqrt(d2 + 1e-8))
    offv = jnp.tanh((xj[:, 49:50] - xi[:, 49:50]) * F32(1.0 / 32.0))

    eh = jnp.dot(dcr, w16_ref[...], preferred_element_type=F32)
    eh = eh + offv * w17_ref[...] + be_ref[...]
    mij = xi[:, 48:49] * xj[:, 48:49]
    eh_ref[0] = eh * mij
    mij_ref[0] = mij


def _encode_edges(Xi_rep, Xj, W16, W17, be2):
    grid = (B, (L * K) // EB)
    return pl.pallas_call(
        _edge_body,
        grid=grid,
        in_specs=[
            pl.BlockSpec((1, EB, TW), lambda b, i: (b, i, 0)),
            pl.BlockSpec((1, EB, TW), lambda b, i: (b, i, 0)),
            pl.BlockSpec((16, DIM_EDGES), lambda b, i: (0, 0)),
            pl.BlockSpec((1, DIM_EDGES), lambda b, i: (0, 0)),
            pl.BlockSpec((1, DIM_EDGES), lambda b, i: (0, 0)),
        ],
        out_specs=[
            pl.BlockSpec((1, EB, DIM_EDGES), lambda b, i: (b, i, 0)),
            pl.BlockSpec((1, EB, 1), lambda b, i: (b, i, 0)),
        ],
        out_shape=[
            jax.ShapeDtypeStruct((B, L * K, DIM_EDGES), F32),
            jax.ShapeDtypeStruct((B, L * K, 1), F32),
        ],
    )(Xi_rep, Xj, W16, W17, be2)


def kernel(X, C, W_node, b_node, W_edge, b_edge):
    Xf = X.reshape(B, L, A * 3)
    Xt = jnp.transpose(Xf, (0, 2, 1))
    C3 = C.reshape(B, L, 1)
    Wn_pad = jnp.concatenate([W_node, jnp.zeros((2, DIM_NODES), F32)], axis=0)
    bn2 = b_node.reshape(1, DIM_NODES)

    eidx32, node_h, mask_i3 = _encode_nodes(Xf, Xt, C3, Wn_pad, bn2)
    edge_idx = eidx32[:, :, :K]
    mask_i = mask_i3.reshape(B, L)

    Xc = jnp.transpose(X, (0, 1, 3, 2))
    ti48 = jnp.broadcast_to(Xc[:, :, :, :, None],
                            (B, L, 3, 4, 4)).reshape(B, L, 48)
    tj48 = jnp.broadcast_to(Xc[:, :, :, None, :],
                            (B, L, 3, 4, 4)).reshape(B, L, 48)
    lvals = jnp.broadcast_to(
        jnp.arange(L, dtype=F32)[None, :, None], (B, L, 1))
    pad = jnp.zeros((B, L, TW - 50), F32)
    ti = jnp.concatenate([ti48, mask_i3, lvals, pad], axis=2)
    tj = jnp.concatenate([tj48, mask_i3, lvals, pad], axis=2)
    gidx = (edge_idx
            + (jnp.arange(B, dtype=jnp.int32) * L)[:, None, None]
            ).reshape(NW, NCH, CHUNK)

    xj = _gather_rows(tj.reshape(B * L, TW), gidx)
    Xj = xj.reshape(B, L * K, TW)
    Xi_rep = jnp.repeat(ti, K, axis=1)

    W16 = W_edge[0:16]
    W17 = W_edge[16:17]
    be2 = b_edge.reshape(1, DIM_EDGES)
    eh_flat, mij_flat = _encode_edges(Xi_rep, Xj, W16, W17, be2)

    edge_h = eh_flat.reshape(B, L, K, DIM_EDGES)
    mask_ij = mij_flat.reshape(B, L, K)
    return node_h, edge_h, edge_idx, mask_i, mask_ij
